# baseline (speedup 1.0000x reference)
import functools

import jax
import jax.numpy as jnp
from jax import lax
from jax.experimental import pallas as pl
from jax.experimental.pallas import tpu as pltpu
from jax.experimental.pallas import tpu_sc as plsc

NC = 2
NS = 16
NW = NC * NS
LANES = 16

B = 4096
LC = 200
LD = 50
EMB = 128
EV = EMB // LANES

BPW = B // NW
LCH = 104
LCV = LC // 2
RLEN = 2 * LCH
NGRP = RLEN // LANES
LDP = 56
OUT_CHUNK = 2

_NEG_INF = float("-inf")


def _score_body(table_ref, w_ref, out_ref):
    out_ref[...] = jnp.sum(table_ref[...] * w_ref[...], axis=1)


def _sc_body(code_ids_hbm, desc_ids_hbm, code_table_hbm, desc_table_hbm,
             svec_hbm, code_out_hbm, desc_out_hbm,
             rows0, rows1, rows2, rows3, drows0, drows1, drows2,
             sc0, sc1, sc2, sc3, cout_v, dout_v,
             cid0, cid1, cid2, cid3, did0, did1, did2, did3,
             csem0, csem1, csem2, csem3, isem0, isem1, isem2, isem3):
    sid = lax.axis_index("s")
    wid = sid * NC + lax.axis_index("c")
    base = pl.multiple_of(wid * BPW, BPW)

    code_bufs = (rows0, rows1, rows2, rows3)
    score_bufs = (sc0, sc1, sc2, sc3)
    code_sems = (csem0, csem1, csem2, csem3)
    desc_bufs = (drows0, drows1, drows2)
    cid = (cid0, cid1, cid2, cid3)
    did = (did0, did1, did2, did3)
    isem = (isem0, isem1, isem2, isem3)

    def fetch_ids(b, q):
        pltpu.make_async_copy(
            code_ids_hbm.at[base + b], cid[q], isem[q]).start()
        pltpu.make_async_copy(
            desc_ids_hbm.at[base + b], did[q], isem[q]).start()

    def wait_ids(q):
        pltpu.make_async_copy(
            code_ids_hbm.at[0], cid[q], isem[q]).wait()
        pltpu.make_async_copy(
            desc_ids_hbm.at[0], did[q], isem[q]).wait()

    def issue_all(q, buf, sbuf, dbuf, sem):
        pltpu.make_async_copy(
            code_table_hbm.at[cid[q].at[0]], buf, sem).start()
        pltpu.make_async_copy(
            svec_hbm.at[cid[q].at[0]], sbuf, sem).start()
        pltpu.make_async_copy(
            desc_table_hbm.at[did[q].at[0]], dbuf, sem).start()

    def wait_cs(buf, sbuf, sem):
        pltpu.make_async_copy(
            code_table_hbm.at[pl.ds(0, RLEN)], buf, sem).wait()
        pltpu.make_async_copy(
            svec_hbm.at[pl.ds(0, RLEN)], sbuf, sem).wait()

    def wait_d(dbuf, sem):
        pltpu.make_async_copy(
            desc_table_hbm.at[pl.ds(0, LDP)], dbuf, sem).wait()

    lanev = lax.iota(jnp.int32, LANES)

    def process_code(b, buf, sbuf):
        def masked(g):
            off = pl.multiple_of(g * LANES, LANES)
            v = sbuf[pl.ds(off, LANES)]
            return jnp.where((off + lanev) % LCH < LCV, v, _NEG_INF)

        def max_body(g, m):
            return jnp.maximum(m, masked(g))
        m = lax.fori_loop(0, NGRP, max_body,
                          jnp.full((LANES,), _NEG_INF, jnp.float32))
        mmax = jnp.max(m)

        def exp_body(g, tot):
            off = pl.multiple_of(g * LANES, LANES)
            e = jnp.exp(masked(g) - mmax)
            sbuf[pl.ds(off, LANES)] = e
            return tot + jnp.sum(e)
        tot = lax.fori_loop(0, NGRP, exp_body, jnp.float32(0.0))
        invv = jnp.ones((LANES,), jnp.float32) / jnp.broadcast_to(tot, (LANES,))

        def scale_body(g, _):
            off = pl.multiple_of(g * LANES, LANES)
            sbuf[pl.ds(off, LANES)] = sbuf[pl.ds(off, LANES)] * invv
            return 0
        lax.fori_loop(0, NGRP, scale_body, 0)

        def body_b(t, acc):
            l0 = 4 * t
            for dj in range(4):
                l = l0 + dj
                wl = plsc.load_gather(sbuf, [jnp.broadcast_to(l, (LANES,))])
                acc = tuple(acc[k] + buf[l, pl.ds(k * LANES, LANES)] * wl
                            for k in range(EV))
            return acc

        acc = tuple(jnp.zeros((LANES,), jnp.float32) for _ in range(EV))
        acc = lax.fori_loop(0, RLEN // 4, body_b, acc)

        slot = lax.rem(b, OUT_CHUNK)
        for k in range(EV):
            cout_v[slot, pl.ds(k * LANES, LANES)] = acc[k]

        @pl.when(slot == OUT_CHUNK - 1)
        def _():
            start = pl.multiple_of(base + b - (OUT_CHUNK - 1), OUT_CHUNK)
            pltpu.sync_copy(cout_v, code_out_hbm.at[pl.ds(start, OUT_CHUNK)])

    def process_desc(b, buf):
        def body_d(l, acc):
            return tuple(acc[k] + buf[l, pl.ds(k * LANES, LANES)]
                         for k in range(EV))
        acc0 = tuple(jnp.zeros((LANES,), jnp.float32) for _ in range(EV))
        acc = lax.fori_loop(0, LD, body_d, acc0)
        scale = 1.0 / LD

        slot = lax.rem(b, OUT_CHUNK)
        for k in range(EV):
            dout_v[slot, pl.ds(k * LANES, LANES)] = acc[k] * scale

        @pl.when(slot == OUT_CHUNK - 1)
        def _():
            start = pl.multiple_of(base + b - (OUT_CHUNK - 1), OUT_CHUNK)
            pltpu.sync_copy(dout_v, desc_out_hbm.at[pl.ds(start, OUT_CHUNK)])

    for q in range(3):
        pltpu.sync_copy(code_ids_hbm.at[base + q], cid[q])
        pltpu.sync_copy(desc_ids_hbm.at[base + q], did[q])
    fetch_ids(3, 3)
    for q in range(3):
        issue_all(q, code_bufs[q], score_bufs[q], desc_bufs[q], code_sems[q])

    nsteps = BPW // 4

    def main_loop(i, _):
        for j in range(4):
            b = 4 * i + j
            dslot = lax.rem(b, 3)

            with jax.named_scope("wait_c"):
                wait_cs(code_bufs[j], score_bufs[j], code_sems[j])
            with jax.named_scope("proc_c"):
                process_code(b, code_bufs[j], score_bufs[j])
            with jax.named_scope("proc_d"):
                for ds in range(3):
                    @pl.when(dslot == ds)
                    def _(ds=ds, j=j, b=b):
                        wait_d(desc_bufs[ds], code_sems[j])
                        process_desc(b, desc_bufs[ds])

            @pl.when(b + 3 < BPW)
            def _(j=j, b=b, dslot=dslot):
                wait_ids((j + 3) % 4)
                for ds in range(3):
                    @pl.when(dslot == ds)
                    def _(ds=ds):
                        issue_all((j + 3) % 4, code_bufs[(j + 3) % 4],
                                  score_bufs[(j + 3) % 4], desc_bufs[ds],
                                  code_sems[(j + 3) % 4])

                @pl.when(b + 4 < BPW)
                def _():
                    fetch_ids(b + 4, j)
        return 0
    lax.fori_loop(0, nsteps, main_loop, 0)


_VB = 4096


@functools.partial(jax.jit, static_argnames=())
def _run(code_ids_pad, desc_ids_pad, code_table, desc_table, attn_w_row):
    vocab = code_table.shape[0]
    ngrid = (vocab + _VB - 1) // _VB
    svec = pl.pallas_call(
        _score_body,
        grid=(ngrid,),
        in_specs=[
            pl.BlockSpec((_VB, EMB), lambda i: (i, 0)),
            pl.BlockSpec((1, EMB), lambda i: (0, 0)),
        ],
        out_specs=pl.BlockSpec((_VB,), lambda i: (i,)),
        out_shape=jax.ShapeDtypeStruct((ngrid * _VB,), jnp.float32),
    )(code_table, attn_w_row)

    mesh = plsc.VectorSubcoreMesh(
        core_axis_name="c", subcore_axis_name="s",
        num_cores=NC, num_subcores=NS)
    fn = pl.kernel(
        _sc_body,
        out_type=(
            jax.ShapeDtypeStruct((B, EMB), jnp.float32),
            jax.ShapeDtypeStruct((B, EMB), jnp.float32),
        ),
        mesh=mesh,
        compiler_params=pltpu.CompilerParams(needs_layout_passes=False),
        scratch_types=(
            pltpu.VMEM((RLEN, EMB), jnp.float32),
            pltpu.VMEM((RLEN, EMB), jnp.float32),
            pltpu.VMEM((RLEN, EMB), jnp.float32),
            pltpu.VMEM((RLEN, EMB), jnp.float32),
            pltpu.VMEM((LDP, EMB), jnp.float32),
            pltpu.VMEM((LDP, EMB), jnp.float32),
            pltpu.VMEM((LDP, EMB), jnp.float32),
            pltpu.VMEM((RLEN,), jnp.float32),
            pltpu.VMEM((RLEN,), jnp.float32),
            pltpu.VMEM((RLEN,), jnp.float32),
            pltpu.VMEM((RLEN,), jnp.float32),
            pltpu.VMEM((OUT_CHUNK, EMB), jnp.float32),
            pltpu.VMEM((OUT_CHUNK, EMB), jnp.float32),
            pltpu.VMEM((1, RLEN), jnp.int32),
            pltpu.VMEM((1, RLEN), jnp.int32),
            pltpu.VMEM((1, RLEN), jnp.int32),
            pltpu.VMEM((1, RLEN), jnp.int32),
            pltpu.VMEM((1, LDP), jnp.int32),
            pltpu.VMEM((1, LDP), jnp.int32),
            pltpu.VMEM((1, LDP), jnp.int32),
            pltpu.VMEM((1, LDP), jnp.int32),
            pltpu.SemaphoreType.DMA,
            pltpu.SemaphoreType.DMA,
            pltpu.SemaphoreType.DMA,
            pltpu.SemaphoreType.DMA,
            pltpu.SemaphoreType.DMA,
            pltpu.SemaphoreType.DMA,
            pltpu.SemaphoreType.DMA,
            pltpu.SemaphoreType.DMA,
        ),
    )
    return fn(code_ids_pad, desc_ids_pad, code_table, desc_table, svec)


def kernel(code_token_ids, code_mask, desc_token_ids, desc_mask,
           code_table, desc_table, attn_w):
    del code_mask, desc_mask
    cids = code_token_ids.astype(jnp.int32).reshape(B, 2, LC // 2)
    cids = jnp.pad(cids, ((0, 0), (0, 0), (0, LCH - LC // 2)))
    cids = cids.reshape(B, 1, RLEN)
    dids = jnp.pad(desc_token_ids.astype(jnp.int32),
                   ((0, 0), (0, LDP - LD))).reshape(B, 1, LDP)
    w = attn_w.reshape(1, EMB).astype(jnp.float32)
    code_pooled, desc_pooled = _run(
        cids, dids, code_table, desc_table, w)
    return (code_pooled, desc_pooled)


# speedup vs baseline: 1.2629x; 1.2629x over previous
import functools

import jax
import jax.numpy as jnp
from jax import lax
from jax.experimental import pallas as pl
from jax.experimental.pallas import tpu as pltpu
from jax.experimental.pallas import tpu_sc as plsc

NC = 2
NS = 16
NW = NC * NS
LANES = 16

B = 4096
LC = 200
LD = 50
EMB = 128
EV = EMB // LANES

BPW = B // NW
RLEN = LC
SLEN = 208
NGRP = SLEN // LANES
LDP = 56
OUT_CHUNK = 2

_NEG_INF = float("-inf")


def _score_body(table_ref, w_ref, out_ref):
    out_ref[...] = jnp.sum(table_ref[...] * w_ref[...], axis=1)


def _sc_body(code_ids_hbm, desc_ids_hbm, code_table_hbm, desc_table_hbm,
             svec_hbm, code_out_hbm, desc_out_hbm,
             rows0, rows1, rows2, rows3, drows0, drows1, drows2,
             sc0, sc1, sc2, sc3, cout_v, dout_v,
             cid0, cid1, cid2, cid3, did0, did1, did2, did3,
             csem0, csem1, csem2, csem3, isem0, isem1, isem2, isem3):
    sid = lax.axis_index("s")
    wid = sid * NC + lax.axis_index("c")
    base = pl.multiple_of(wid * BPW, BPW)

    code_bufs = (rows0, rows1, rows2, rows3)
    score_bufs = (sc0, sc1, sc2, sc3)
    code_sems = (csem0, csem1, csem2, csem3)
    desc_bufs = (drows0, drows1, drows2)
    cid = (cid0, cid1, cid2, cid3)
    did = (did0, did1, did2, did3)
    isem = (isem0, isem1, isem2, isem3)

    def fetch_ids(b, q):
        pltpu.make_async_copy(
            code_ids_hbm.at[base + b], cid[q], isem[q]).start()
        pltpu.make_async_copy(
            desc_ids_hbm.at[base + b], did[q], isem[q]).start()

    def wait_ids(q):
        pltpu.make_async_copy(
            code_ids_hbm.at[0], cid[q], isem[q]).wait()
        pltpu.make_async_copy(
            desc_ids_hbm.at[0], did[q], isem[q]).wait()

    def issue_all(q, buf, sbuf, dbuf, sem):
        pltpu.make_async_copy(
            code_table_hbm.at[cid[q].at[0]], buf, sem).start()
        pltpu.make_async_copy(
            svec_hbm.at[cid[q].at[0]], sbuf.at[pl.ds(0, RLEN)], sem).start()
        pltpu.make_async_copy(
            desc_table_hbm.at[did[q].at[0]], dbuf, sem).start()

    def wait_cs(buf, sbuf, sem):
        pltpu.make_async_copy(
            code_table_hbm.at[pl.ds(0, RLEN)], buf, sem).wait()
        pltpu.make_async_copy(
            svec_hbm.at[pl.ds(0, RLEN)], sbuf.at[pl.ds(0, RLEN)], sem).wait()

    def wait_d(dbuf, sem):
        pltpu.make_async_copy(
            desc_table_hbm.at[pl.ds(0, LDP)], dbuf, sem).wait()

    lanev = lax.iota(jnp.int32, LANES)

    def process_code(b, buf, sbuf):
        def masked(g):
            off = pl.multiple_of(g * LANES, LANES)
            v = sbuf[pl.ds(off, LANES)]
            return jnp.where(off + lanev < RLEN, v, _NEG_INF)

        def max_body(g, m):
            return jnp.maximum(m, masked(g))
        m = lax.fori_loop(0, NGRP, max_body,
                          jnp.full((LANES,), _NEG_INF, jnp.float32))
        mmax = jnp.max(m)

        def exp_body(g, tot):
            off = pl.multiple_of(g * LANES, LANES)
            e = jnp.exp(masked(g) - mmax)
            sbuf[pl.ds(off, LANES)] = e
            return tot + jnp.sum(e)
        tot = lax.fori_loop(0, NGRP, exp_body, jnp.float32(0.0))
        invv = jnp.ones((LANES,), jnp.float32) / jnp.broadcast_to(tot, (LANES,))

        def scale_body(g, _):
            off = pl.multiple_of(g * LANES, LANES)
            sbuf[pl.ds(off, LANES)] = sbuf[pl.ds(off, LANES)] * invv
            return 0
        lax.fori_loop(0, NGRP, scale_body, 0)

        def body_b(t, acc):
            l0 = 4 * t
            for dj in range(4):
                l = l0 + dj
                wl = plsc.load_gather(sbuf, [jnp.broadcast_to(l, (LANES,))])
                acc = tuple(acc[k] + buf[l, pl.ds(k * LANES, LANES)] * wl
                            for k in range(EV))
            return acc

        acc = tuple(jnp.zeros((LANES,), jnp.float32) for _ in range(EV))
        acc = lax.fori_loop(0, RLEN // 4, body_b, acc)

        slot = lax.rem(b, OUT_CHUNK)
        for k in range(EV):
            cout_v[slot, pl.ds(k * LANES, LANES)] = acc[k]

        @pl.when(slot == OUT_CHUNK - 1)
        def _():
            start = pl.multiple_of(base + b - (OUT_CHUNK - 1), OUT_CHUNK)
            pltpu.sync_copy(cout_v, code_out_hbm.at[pl.ds(start, OUT_CHUNK)])

    def process_desc(b, buf):
        def body_d(l, acc):
            return tuple(acc[k] + buf[l, pl.ds(k * LANES, LANES)]
                         for k in range(EV))
        acc0 = tuple(jnp.zeros((LANES,), jnp.float32) for _ in range(EV))
        acc = lax.fori_loop(0, LD, body_d, acc0)
        scale = 1.0 / LD

        slot = lax.rem(b, OUT_CHUNK)
        for k in range(EV):
            dout_v[slot, pl.ds(k * LANES, LANES)] = acc[k] * scale

        @pl.when(slot == OUT_CHUNK - 1)
        def _():
            start = pl.multiple_of(base + b - (OUT_CHUNK - 1), OUT_CHUNK)
            pltpu.sync_copy(dout_v, desc_out_hbm.at[pl.ds(start, OUT_CHUNK)])

    for q in range(3):
        pltpu.sync_copy(code_ids_hbm.at[base + q], cid[q])
        pltpu.sync_copy(desc_ids_hbm.at[base + q], did[q])
    fetch_ids(3, 3)
    for q in range(3):
        issue_all(q, code_bufs[q], score_bufs[q], desc_bufs[q], code_sems[q])

    nsteps = BPW // 4

    def main_loop(i, _):
        for j in range(4):
            b = 4 * i + j
            dslot = lax.rem(b, 3)

            with jax.named_scope("wait_c"):
                wait_cs(code_bufs[j], score_bufs[j], code_sems[j])
            with jax.named_scope("proc_c"):
                process_code(b, code_bufs[j], score_bufs[j])
            with jax.named_scope("proc_d"):
                for ds in range(3):
                    @pl.when(dslot == ds)
                    def _(ds=ds, j=j, b=b):
                        wait_d(desc_bufs[ds], code_sems[j])
                        process_desc(b, desc_bufs[ds])

            @pl.when(b + 3 < BPW)
            def _(j=j, b=b, dslot=dslot):
                wait_ids((j + 3) % 4)
                for ds in range(3):
                    @pl.when(dslot == ds)
                    def _(ds=ds):
                        issue_all((j + 3) % 4, code_bufs[(j + 3) % 4],
                                  score_bufs[(j + 3) % 4], desc_bufs[ds],
                                  code_sems[(j + 3) % 4])

                @pl.when(b + 4 < BPW)
                def _():
                    fetch_ids(b + 4, j)
        return 0
    lax.fori_loop(0, nsteps, main_loop, 0)


_VB = 4096


@functools.partial(jax.jit, static_argnames=())
def _run(code_ids_pad, desc_ids_pad, code_table, desc_table, attn_w_row):
    vocab = code_table.shape[0]
    ngrid = (vocab + _VB - 1) // _VB
    svec = pl.pallas_call(
        _score_body,
        grid=(ngrid,),
        in_specs=[
            pl.BlockSpec((_VB, EMB), lambda i: (i, 0)),
            pl.BlockSpec((1, EMB), lambda i: (0, 0)),
        ],
        out_specs=pl.BlockSpec((_VB,), lambda i: (i,)),
        out_shape=jax.ShapeDtypeStruct((ngrid * _VB,), jnp.float32),
    )(code_table, attn_w_row)

    mesh = plsc.VectorSubcoreMesh(
        core_axis_name="c", subcore_axis_name="s",
        num_cores=NC, num_subcores=NS)
    fn = pl.kernel(
        _sc_body,
        out_type=(
            jax.ShapeDtypeStruct((B, EMB), jnp.float32),
            jax.ShapeDtypeStruct((B, EMB), jnp.float32),
        ),
        mesh=mesh,
        compiler_params=pltpu.CompilerParams(needs_layout_passes=False),
        scratch_types=(
            pltpu.VMEM((RLEN, EMB), jnp.float32),
            pltpu.VMEM((RLEN, EMB), jnp.float32),
            pltpu.VMEM((RLEN, EMB), jnp.float32),
            pltpu.VMEM((RLEN, EMB), jnp.float32),
            pltpu.VMEM((LDP, EMB), jnp.float32),
            pltpu.VMEM((LDP, EMB), jnp.float32),
            pltpu.VMEM((LDP, EMB), jnp.float32),
            pltpu.VMEM((SLEN,), jnp.float32),
            pltpu.VMEM((SLEN,), jnp.float32),
            pltpu.VMEM((SLEN,), jnp.float32),
            pltpu.VMEM((SLEN,), jnp.float32),
            pltpu.VMEM((OUT_CHUNK, EMB), jnp.float32),
            pltpu.VMEM((OUT_CHUNK, EMB), jnp.float32),
            pltpu.VMEM((1, RLEN), jnp.int32),
            pltpu.VMEM((1, RLEN), jnp.int32),
            pltpu.VMEM((1, RLEN), jnp.int32),
            pltpu.VMEM((1, RLEN), jnp.int32),
            pltpu.VMEM((1, LDP), jnp.int32),
            pltpu.VMEM((1, LDP), jnp.int32),
            pltpu.VMEM((1, LDP), jnp.int32),
            pltpu.VMEM((1, LDP), jnp.int32),
            pltpu.SemaphoreType.DMA,
            pltpu.SemaphoreType.DMA,
            pltpu.SemaphoreType.DMA,
            pltpu.SemaphoreType.DMA,
            pltpu.SemaphoreType.DMA,
            pltpu.SemaphoreType.DMA,
            pltpu.SemaphoreType.DMA,
            pltpu.SemaphoreType.DMA,
        ),
    )
    return fn(code_ids_pad, desc_ids_pad, code_table, desc_table, svec)


def kernel(code_token_ids, code_mask, desc_token_ids, desc_mask,
           code_table, desc_table, attn_w):
    del code_mask, desc_mask
    cids = code_token_ids.astype(jnp.int32).reshape(B, 1, RLEN)
    dids = jnp.pad(desc_token_ids.astype(jnp.int32),
                   ((0, 0), (0, LDP - LD))).reshape(B, 1, LDP)
    w = attn_w.reshape(1, EMB).astype(jnp.float32)
    code_pooled, desc_pooled = _run(
        cids, dids, code_table, desc_table, w)
    return (code_pooled, desc_pooled)


# speedup vs baseline: 3.4944x; 2.7670x over previous
import functools

import jax
import jax.numpy as jnp
from jax import lax
from jax.experimental import pallas as pl
from jax.experimental.pallas import tpu as pltpu
from jax.experimental.pallas import tpu_sc as plsc

NC = 2
NS = 16
NW = NC * NS
LANES = 16

B = 4096
LC = 200
LD = 50
EMB = 128
EV = EMB // LANES

BPW = B // NW
RLEN = LC
SLEN = 208
NGRP = SLEN // LANES
LDP = 56
OUT_CHUNK = 2

_NEG_INF = float("-inf")


def _score_body(table_ref, w_ref, out_ref):
    out_ref[...] = jnp.sum(table_ref[...] * w_ref[...], axis=1)


def _sc_body(code_ids_hbm, desc_ids_hbm, code_table_hbm, desc_table_hbm,
             svec_hbm, code_out_hbm, desc_out_hbm,
             rows0, rows1, rows2, rows3, drows0, drows1, drows2,
             sc0, sc1, sc2, sc3, cout_v, dout_v,
             cid0, cid1, cid2, cid3, did0, did1, did2, did3,
             csem0, csem1, csem2, csem3, isem0, isem1, isem2, isem3):
    sid = lax.axis_index("s")
    wid = sid * NC + lax.axis_index("c")
    base = pl.multiple_of(wid * BPW, BPW)

    code_bufs = (rows0, rows1, rows2, rows3)
    score_bufs = (sc0, sc1, sc2, sc3)
    code_sems = (csem0, csem1, csem2, csem3)
    desc_bufs = (drows0, drows1, drows2)
    cid = (cid0, cid1, cid2, cid3)
    did = (did0, did1, did2, did3)
    isem = (isem0, isem1, isem2, isem3)

    def fetch_ids(b, q):
        pltpu.make_async_copy(
            code_ids_hbm.at[base + b], cid[q], isem[q]).start()
        pltpu.make_async_copy(
            desc_ids_hbm.at[base + b], did[q], isem[q]).start()

    def wait_ids(q):
        pltpu.make_async_copy(
            code_ids_hbm.at[0], cid[q], isem[q]).wait()
        pltpu.make_async_copy(
            desc_ids_hbm.at[0], did[q], isem[q]).wait()

    def issue_all(q, buf, sbuf, dbuf, sem):
        pltpu.make_async_copy(
            code_table_hbm.at[cid[q].at[0]], buf, sem).start()
        pltpu.make_async_copy(
            svec_hbm.at[cid[q].at[0]], sbuf.at[pl.ds(0, RLEN)], sem).start()
        pltpu.make_async_copy(
            desc_table_hbm.at[did[q].at[0]], dbuf, sem).start()

    def wait_cs(buf, sbuf, sem):
        pltpu.make_async_copy(
            code_table_hbm.at[pl.ds(0, RLEN)], buf, sem).wait()
        pltpu.make_async_copy(
            svec_hbm.at[pl.ds(0, RLEN)], sbuf.at[pl.ds(0, RLEN)], sem).wait()

    def wait_d(dbuf, sem):
        pltpu.make_async_copy(
            desc_table_hbm.at[pl.ds(0, LDP)], dbuf, sem).wait()

    lanev = lax.iota(jnp.int32, LANES)

    def process_code(b, buf, sbuf):
        def masked(g):
            off = pl.multiple_of(g * LANES, LANES)
            v = sbuf[pl.ds(off, LANES)]
            return jnp.where(off + lanev < RLEN, v, _NEG_INF)

        def max_body(g, m):
            return jnp.maximum(m, masked(g))
        m = lax.fori_loop(0, NGRP, max_body,
                          jnp.full((LANES,), _NEG_INF, jnp.float32))
        mmax = jnp.max(m)

        def exp_body(g, tot):
            off = pl.multiple_of(g * LANES, LANES)
            e = jnp.exp(masked(g) - mmax)
            sbuf[pl.ds(off, LANES)] = e
            return tot + jnp.sum(e)
        tot = lax.fori_loop(0, NGRP, exp_body, jnp.float32(0.0))
        invv = jnp.ones((LANES,), jnp.float32) / jnp.broadcast_to(tot, (LANES,))

        def scale_body(g, _):
            off = pl.multiple_of(g * LANES, LANES)
            sbuf[pl.ds(off, LANES)] = sbuf[pl.ds(off, LANES)] * invv
            return 0
        lax.fori_loop(0, NGRP, scale_body, 0)

        def body_b(t, acc):
            l0 = 4 * t
            for dj in range(4):
                l = l0 + dj
                wl = plsc.load_gather(sbuf, [jnp.broadcast_to(l, (LANES,))])
                acc = tuple(acc[k] + buf[l, pl.ds(k * LANES, LANES)] * wl
                            for k in range(EV))
            return acc

        acc = tuple(jnp.zeros((LANES,), jnp.float32) for _ in range(EV))
        acc = lax.fori_loop(0, RLEN // 4, body_b, acc)

        slot = lax.rem(b, OUT_CHUNK)
        for k in range(EV):
            cout_v[slot, pl.ds(k * LANES, LANES)] = acc[k]

        @pl.when(slot == OUT_CHUNK - 1)
        def _():
            start = pl.multiple_of(base + b - (OUT_CHUNK - 1), OUT_CHUNK)
            pltpu.sync_copy(cout_v, code_out_hbm.at[pl.ds(start, OUT_CHUNK)])

    def process_desc(b, buf):
        def body_d(l, acc):
            return tuple(acc[k] + buf[l, pl.ds(k * LANES, LANES)]
                         for k in range(EV))
        acc0 = tuple(jnp.zeros((LANES,), jnp.float32) for _ in range(EV))
        acc = lax.fori_loop(0, LD, body_d, acc0)
        scale = 1.0 / LD

        slot = lax.rem(b, OUT_CHUNK)
        for k in range(EV):
            dout_v[slot, pl.ds(k * LANES, LANES)] = acc[k] * scale

        @pl.when(slot == OUT_CHUNK - 1)
        def _():
            start = pl.multiple_of(base + b - (OUT_CHUNK - 1), OUT_CHUNK)
            pltpu.sync_copy(dout_v, desc_out_hbm.at[pl.ds(start, OUT_CHUNK)])

    for q in range(3):
        pltpu.sync_copy(code_ids_hbm.at[base + q], cid[q])
        pltpu.sync_copy(desc_ids_hbm.at[base + q], did[q])
    fetch_ids(3, 3)
    for q in range(3):
        issue_all(q, code_bufs[q], score_bufs[q], desc_bufs[q], code_sems[q])

    nsteps = BPW // 4

    def main_loop(i, _):
        for j in range(4):
            b = 4 * i + j
            dslot = lax.rem(b, 3)

            with jax.named_scope("wait_c"):
                wait_cs(code_bufs[j], score_bufs[j], code_sems[j])
            with jax.named_scope("proc_c"):
                process_code(b, code_bufs[j], score_bufs[j])
            with jax.named_scope("proc_d"):
                for ds in range(3):
                    @pl.when(dslot == ds)
                    def _(ds=ds, j=j, b=b):
                        wait_d(desc_bufs[ds], code_sems[j])
                        process_desc(b, desc_bufs[ds])

            @pl.when(b + 3 < BPW)
            def _(j=j, b=b, dslot=dslot):
                wait_ids((j + 3) % 4)
                for ds in range(3):
                    @pl.when(dslot == ds)
                    def _(ds=ds):
                        issue_all((j + 3) % 4, code_bufs[(j + 3) % 4],
                                  score_bufs[(j + 3) % 4], desc_bufs[ds],
                                  code_sems[(j + 3) % 4])

                @pl.when(b + 4 < BPW)
                def _():
                    fetch_ids(b + 4, j)
        return 0
    lax.fori_loop(0, nsteps, main_loop, 0)


_VB = 4096


@functools.partial(jax.jit, static_argnames=())
def _run(code_ids_pad, desc_ids_pad, code_table, desc_table, attn_w_row):
    vocab = code_table.shape[0]
    ngrid = (vocab + _VB - 1) // _VB
    svec = pl.pallas_call(
        _score_body,
        grid=(ngrid,),
        in_specs=[
            pl.BlockSpec((_VB, EMB), lambda i: (i, 0)),
            pl.BlockSpec((1, EMB), lambda i: (0, 0)),
        ],
        out_specs=pl.BlockSpec((_VB,), lambda i: (i,)),
        out_shape=jax.ShapeDtypeStruct((ngrid * _VB,), jnp.float32),
    )(code_table, attn_w_row)

    mesh = plsc.VectorSubcoreMesh(
        core_axis_name="c", subcore_axis_name="s",
        num_cores=NC, num_subcores=NS)
    fn = pl.kernel(
        _sc_body,
        out_type=(
            jax.ShapeDtypeStruct((B, EMB), jnp.float32),
            jax.ShapeDtypeStruct((B, EMB), jnp.float32),
        ),
        mesh=mesh,
        compiler_params=pltpu.CompilerParams(needs_layout_passes=False),
        scratch_types=(
            pltpu.VMEM((RLEN, EMB), jnp.float32),
            pltpu.VMEM((RLEN, EMB), jnp.float32),
            pltpu.VMEM((RLEN, EMB), jnp.float32),
            pltpu.VMEM((RLEN, EMB), jnp.float32),
            pltpu.VMEM((LDP, EMB), jnp.float32),
            pltpu.VMEM((LDP, EMB), jnp.float32),
            pltpu.VMEM((LDP, EMB), jnp.float32),
            pltpu.VMEM((SLEN,), jnp.float32),
            pltpu.VMEM((SLEN,), jnp.float32),
            pltpu.VMEM((SLEN,), jnp.float32),
            pltpu.VMEM((SLEN,), jnp.float32),
            pltpu.VMEM((OUT_CHUNK, EMB), jnp.float32),
            pltpu.VMEM((OUT_CHUNK, EMB), jnp.float32),
            pltpu.VMEM((1, RLEN), jnp.int32),
            pltpu.VMEM((1, RLEN), jnp.int32),
            pltpu.VMEM((1, RLEN), jnp.int32),
            pltpu.VMEM((1, RLEN), jnp.int32),
            pltpu.VMEM((1, LDP), jnp.int32),
            pltpu.VMEM((1, LDP), jnp.int32),
            pltpu.VMEM((1, LDP), jnp.int32),
            pltpu.VMEM((1, LDP), jnp.int32),
            pltpu.SemaphoreType.DMA,
            pltpu.SemaphoreType.DMA,
            pltpu.SemaphoreType.DMA,
            pltpu.SemaphoreType.DMA,
            pltpu.SemaphoreType.DMA,
            pltpu.SemaphoreType.DMA,
            pltpu.SemaphoreType.DMA,
            pltpu.SemaphoreType.DMA,
        ),
    )
    return fn(code_ids_pad, desc_ids_pad, code_table, desc_table, svec)


def kernel(code_token_ids, code_mask, desc_token_ids, desc_mask,
           code_table, desc_table, attn_w):
    del code_mask, desc_mask
    cids = code_token_ids.astype(jnp.int32).reshape(B, 1, RLEN)
    dpad = jnp.broadcast_to(jnp.arange(LDP - LD, dtype=jnp.int32),
                            (B, LDP - LD))
    dids = jnp.concatenate(
        [desc_token_ids.astype(jnp.int32), dpad], axis=1).reshape(B, 1, LDP)
    w = attn_w.reshape(1, EMB).astype(jnp.float32)
    code_pooled, desc_pooled = _run(
        cids, dids, code_table, desc_table, w)
    return (code_pooled, desc_pooled)
